# baseline (device time: 310303 ns/iter reference)
import jax
import jax.numpy as jnp
from jax import lax
from jax.experimental import pallas as pl
from jax.experimental.pallas import tpu as pltpu

N_DEV = 32


def _gelu(y):
    c = 0.7978845608028654
    return 0.5 * y * (1.0 + jnp.tanh(c * (y + 0.044715 * y**3)))


def kernel(x, w_mat):
    m, _ = x.shape
    _, n = w_mat.shape
    ch = m // N_DEV

    def body(x_ref, w_ref, out_ref, rs_buf, ag_buf, stage, own_buf,
             rs_send, rs_recv, ag_send, ag_recv):
        me = lax.axis_index("i")
        left = lax.rem(me + (N_DEV - 1), N_DEV)
        right = lax.rem(me + 1, N_DEV)

        barrier = pltpu.get_barrier_semaphore()
        for nbr in (left, right):
            pl.semaphore_signal(barrier, inc=1, device_id=(nbr,),
                                device_id_type=pl.DeviceIdType.MESH)
        pl.semaphore_wait(barrier, 2)

        w = w_ref[:, :]

        def partial_chunk(c):
            xs = x_ref[pl.ds(c * ch, ch), :]
            return jnp.dot(xs, w, preferred_element_type=jnp.float32)

        def mod(a):
            return lax.rem(a, N_DEV)

        stage[0] = partial_chunk(me).astype(jnp.bfloat16)
        acc = None
        for t in range(N_DEV - 1):
            rdma = pltpu.make_async_remote_copy(
                src_ref=stage.at[t % 2],
                dst_ref=rs_buf.at[t],
                send_sem=rs_send.at[t],
                recv_sem=rs_recv.at[t],
                device_id=(right,),
                device_id_type=pl.DeviceIdType.MESH,
            )
            rdma.start()
            p = partial_chunk(mod(me + (2 * N_DEV - t - 1)))
            rdma.wait_recv()
            acc = p + rs_buf[t].astype(jnp.float32)
            if t < N_DEV - 2:
                stage[(t + 1) % 2] = acc.astype(jnp.bfloat16)
            rdma.wait_send()

        owned = mod(me + 1)
        g = _gelu(acc)
        out_ref[pl.ds(owned * ch, ch), :] = g
        own_buf[:, :] = g.astype(jnp.bfloat16)

        for h in range(N_DEV - 1):
            src = own_buf if h == 0 else ag_buf.at[h - 1]
            rdma = pltpu.make_async_remote_copy(
                src_ref=src,
                dst_ref=ag_buf.at[h],
                send_sem=ag_send.at[h],
                recv_sem=ag_recv.at[h],
                device_id=(right,),
                device_id_type=pl.DeviceIdType.MESH,
            )
            rdma.start()
            rdma.wait_recv()
            origin = mod(me + (N_DEV - h))
            out_ref[pl.ds(origin * ch, ch), :] = ag_buf[h].astype(jnp.float32)
            rdma.wait_send()

    return pl.pallas_call(
        body,
        out_shape=jax.ShapeDtypeStruct((m, n), jnp.float32),
        in_specs=[
            pl.BlockSpec(memory_space=pltpu.VMEM),
            pl.BlockSpec(memory_space=pltpu.VMEM),
        ],
        out_specs=pl.BlockSpec(memory_space=pltpu.VMEM),
        scratch_shapes=[
            pltpu.VMEM((N_DEV - 1, ch, n), jnp.bfloat16),
            pltpu.VMEM((N_DEV - 1, ch, n), jnp.bfloat16),
            pltpu.VMEM((2, ch, n), jnp.bfloat16),
            pltpu.VMEM((ch, n), jnp.bfloat16),
            pltpu.SemaphoreType.DMA((N_DEV - 1,)),
            pltpu.SemaphoreType.DMA((N_DEV - 1,)),
            pltpu.SemaphoreType.DMA((N_DEV - 1,)),
            pltpu.SemaphoreType.DMA((N_DEV - 1,)),
        ],
        compiler_params=pltpu.CompilerParams(collective_id=0),
    )(x, w_mat)


# device time: 231969 ns/iter; 1.3377x vs baseline; 1.3377x over previous
import jax
import jax.numpy as jnp
from jax import lax
from jax.experimental import pallas as pl
from jax.experimental.pallas import tpu as pltpu

N_DEV = 32
MESH = pl.DeviceIdType.MESH


def _gelu(y):
    c = 0.7978845608028654
    return 0.5 * y * (1.0 + jnp.tanh(c * (y + 0.044715 * y**3)))


def kernel(x, w_mat):
    m, _ = x.shape
    _, n = w_mat.shape
    hc = n // 2
    r1, r2, r3 = m // 4, m // 16, m // 32
    f32, bf16 = jnp.float32, jnp.bfloat16

    def body(x_ref, w_ref, out_ref,
             zstA, zstB, zrsA, zrsB, ystA, ystB, yrsA, yrsB,
             xst, xrs, own_buf, xag, yblkA, yblkB, yagA, yagB,
             zsrcA, zsrcB, zagA, zagB, acczA, acczB, accyA, accyB,
             zssA, zrrA, zssB, zrrB, yssA, yrrA, yssB, yrrB,
             xss, xrr, gxss, gxrr,
             gyssA, gyrrA, gyssB, gyrrB, gzssA, gzrrA, gzssB, gzrrB):
        me = lax.axis_index("i")
        z_ = me // 8
        q = lax.rem(me, 8)
        y_ = q // 2
        r = lax.rem(q, 2)
        x_ = jnp.where(lax.rem(y_, 2) == 0, r, 1 - r)

        def mod4(a):
            return lax.rem(a, 4)

        def idx(xx, yy, zz):
            return 8 * zz + 2 * yy + jnp.where(lax.rem(yy, 2) == 0, xx, 1 - xx)

        zf = idx(x_, y_, mod4(z_ + 1))
        zb = idx(x_, y_, mod4(z_ + 3))
        yf = idx(x_, mod4(y_ + 1), z_)
        yb = idx(x_, mod4(y_ + 3), z_)
        xp = idx(1 - x_, y_, z_)

        barrier = pltpu.get_barrier_semaphore()
        for nbr in (zf, zb, yf, yb, xp):
            pl.semaphore_signal(barrier, inc=1, device_id=(nbr,),
                                device_id_type=MESH)
        pl.semaphore_wait(barrier, 5)

        wA = w_ref[:, 0:hc]
        wB = w_ref[:, hc:n]

        def remote(src, dst, ss, rr, dev):
            return pltpu.make_async_remote_copy(
                src_ref=src, dst_ref=dst, send_sem=ss, recv_sem=rr,
                device_id=(dev,), device_id_type=MESH)

        def bidir_rs(pos, partial_fn, stA, stB, bufA, bufB,
                     ssA, rrA, ssB, rrB, fwd, bwd):
            stA[0] = partial_fn(mod4(pos + 3), 0).astype(bf16)
            stB[0] = partial_fn(mod4(pos + 1), 1).astype(bf16)
            accA = accB = None
            for t in range(3):
                rdA = remote(stA.at[t % 2], bufA.at[t], ssA.at[t], rrA.at[t], fwd)
                rdB = remote(stB.at[t % 2], bufB.at[t], ssB.at[t], rrB.at[t], bwd)
                rdA.start()
                rdB.start()
                pA = partial_fn(mod4(pos + 2 - t + 4), 0)
                pB = partial_fn(mod4(pos + 2 + t), 1)
                rdA.wait_recv()
                accA = pA + bufA[t].astype(f32)
                if t < 2:
                    stA[(t + 1) % 2] = accA.astype(bf16)
                rdB.wait_recv()
                accB = pB + bufB[t].astype(f32)
                if t < 2:
                    stB[(t + 1) % 2] = accB.astype(bf16)
                rdA.wait_send()
                rdB.wait_send()
            return accA, accB

        def bidir_ag(pos, ownA, ownB, agA, agB, ssA, rrA, ssB, rrB,
                     fwd, bwd, store):
            for h in range(3):
                srcA = ownA if h == 0 else agA.at[h - 1]
                srcB = ownB if h == 0 else agB.at[h - 1]
                rdA = remote(srcA, agA.at[h], ssA.at[h], rrA.at[h], fwd)
                rdB = remote(srcB, agB.at[h], ssB.at[h], rrB.at[h], bwd)
                rdA.start()
                rdB.start()
                rdA.wait_recv()
                store(0, mod4(pos + 3 - h), agA[h])
                rdB.wait_recv()
                store(1, mod4(pos + 1 + h), agB[h])
                rdA.wait_send()
                rdB.wait_send()

        def pz(g, half):
            xs = x_ref[pl.ds(g * r1, r1), :]
            return jnp.dot(xs, wA if half == 0 else wB,
                           preferred_element_type=f32)

        zA, zB = bidir_rs(z_, pz, zstA, zstB, zrsA, zrsB,
                          zssA, zrrA, zssB, zrrB, zf, zb)
        acczA[:, :] = zA
        acczB[:, :] = zB

        def py(g, half):
            acc = acczA if half == 0 else acczB
            return acc[pl.ds(g * r2, r2), :]

        yA, yB = bidir_rs(y_, py, ystA, ystB, yrsA, yrsB,
                          yssA, yrrA, yssB, yrrB, yf, yb)
        accyA[:, :] = yA
        accyB[:, :] = yB

        xst[:, 0:hc] = accyA[pl.ds((1 - x_) * r3, r3), :].astype(bf16)
        xst[:, hc:n] = accyB[pl.ds((1 - x_) * r3, r3), :].astype(bf16)
        rdx = remote(xst, xrs, xss, xrr, xp)
        rdx.start()
        mine = jnp.concatenate(
            [accyA[pl.ds(x_ * r3, r3), :], accyB[pl.ds(x_ * r3, r3), :]],
            axis=1)
        rdx.wait_recv()
        final = mine + xrs[:, :].astype(f32)
        rdx.wait_send()

        g64 = _gelu(final)
        r0 = r1 * z_ + r2 * y_ + r3 * x_
        out_ref[pl.ds(r0, r3), :] = g64
        own_buf[:, :] = g64.astype(bf16)

        rdg = remote(own_buf, xag, gxss, gxrr, xp)
        rdg.start()
        rdg.wait_recv()
        r0p = r1 * z_ + r2 * y_ + r3 * (1 - x_)
        out_ref[pl.ds(r0p, r3), :] = xag[:, :].astype(f32)
        rdg.wait_send()

        yblkA[pl.ds(x_ * r3, r3), :] = own_buf[:, 0:hc]
        yblkA[pl.ds((1 - x_) * r3, r3), :] = xag[:, 0:hc]
        yblkB[pl.ds(x_ * r3, r3), :] = own_buf[:, hc:n]
        yblkB[pl.ds((1 - x_) * r3, r3), :] = xag[:, hc:n]
        zsrcA[pl.ds(y_ * r2, r2), :] = yblkA[:, :]
        zsrcB[pl.ds(y_ * r2, r2), :] = yblkB[:, :]

        def store_y(half, g, data):
            row = r1 * z_ + r2 * g
            if half == 0:
                out_ref[pl.ds(row, r2), 0:hc] = data.astype(f32)
                zsrcA[pl.ds(g * r2, r2), :] = data
            else:
                out_ref[pl.ds(row, r2), hc:n] = data.astype(f32)
                zsrcB[pl.ds(g * r2, r2), :] = data

        bidir_ag(y_, yblkA, yblkB, yagA, yagB,
                 gyssA, gyrrA, gyssB, gyrrB, yf, yb, store_y)

        def store_z(half, g, data):
            if half == 0:
                out_ref[pl.ds(g * r1, r1), 0:hc] = data.astype(f32)
            else:
                out_ref[pl.ds(g * r1, r1), hc:n] = data.astype(f32)

        bidir_ag(z_, zsrcA, zsrcB, zagA, zagB,
                 gzssA, gzrrA, gzssB, gzrrB, zf, zb, store_z)

    dma3 = pltpu.SemaphoreType.DMA((3,))
    dma1 = pltpu.SemaphoreType.DMA
    return pl.pallas_call(
        body,
        out_shape=jax.ShapeDtypeStruct((m, n), jnp.float32),
        in_specs=[
            pl.BlockSpec(memory_space=pltpu.VMEM),
            pl.BlockSpec(memory_space=pltpu.VMEM),
        ],
        out_specs=pl.BlockSpec(memory_space=pltpu.VMEM),
        scratch_shapes=[
            pltpu.VMEM((2, r1, hc), bf16),
            pltpu.VMEM((2, r1, hc), bf16),
            pltpu.VMEM((3, r1, hc), bf16),
            pltpu.VMEM((3, r1, hc), bf16),
            pltpu.VMEM((2, r2, hc), bf16),
            pltpu.VMEM((2, r2, hc), bf16),
            pltpu.VMEM((3, r2, hc), bf16),
            pltpu.VMEM((3, r2, hc), bf16),
            pltpu.VMEM((r3, n), bf16),
            pltpu.VMEM((r3, n), bf16),
            pltpu.VMEM((r3, n), bf16),
            pltpu.VMEM((r3, n), bf16),
            pltpu.VMEM((r2, hc), bf16),
            pltpu.VMEM((r2, hc), bf16),
            pltpu.VMEM((3, r2, hc), bf16),
            pltpu.VMEM((3, r2, hc), bf16),
            pltpu.VMEM((r1, hc), bf16),
            pltpu.VMEM((r1, hc), bf16),
            pltpu.VMEM((3, r1, hc), bf16),
            pltpu.VMEM((3, r1, hc), bf16),
            pltpu.VMEM((r1, hc), f32),
            pltpu.VMEM((r1, hc), f32),
            pltpu.VMEM((r2, hc), f32),
            pltpu.VMEM((r2, hc), f32),
            dma3, dma3, dma3, dma3,
            dma3, dma3, dma3, dma3,
            dma1, dma1, dma1, dma1,
            dma3, dma3, dma3, dma3,
            dma3, dma3, dma3, dma3,
        ],
        compiler_params=pltpu.CompilerParams(
            collective_id=0, vmem_limit_bytes=64 * 1024 * 1024),
    )(x, w_mat)


# device time: 165993 ns/iter; 1.8694x vs baseline; 1.3975x over previous
import jax
import jax.numpy as jnp
from jax import lax
from jax.experimental import pallas as pl
from jax.experimental.pallas import tpu as pltpu

MESH = pl.DeviceIdType.MESH


def _gelu(y):
    c = 0.7978845608028654
    return 0.5 * y * (1.0 + jnp.tanh(c * (y + 0.044715 * y**3)))


def kernel(x, w_mat):
    m, _ = x.shape
    _, n = w_mat.shape
    hc = n // 2
    rp, rz = m // 8, m // 32
    f32, bf16 = jnp.float32, jnp.bfloat16

    def body(x_ref, w_ref, out_ref,
             pstA, pstB, prsA, prsB, zstA, zstB, zrsA, zrsB,
             planeA, planeB, own_buf, ownA, ownB,
             zagA, zagB, psrcA, psrcB, pagA, pagB,
             pssA, prrA, pssB, prrB, zssA, zrrA, zssB, zrrB,
             gzssA, gzrrA, gzssB, gzrrB, gpssA, gprrA, gpssB, gprrB):
        me = lax.axis_index("i")
        z_ = me // 8
        q = lax.rem(me, 8)
        y_ = q // 2
        r = lax.rem(q, 2)
        x_ = jnp.where(lax.rem(y_, 2) == 0, r, 1 - r)
        p_ = jnp.where(x_ == 0, y_, 7 - y_)

        def idx(xx, yy, zz):
            return 8 * zz + 2 * yy + jnp.where(lax.rem(yy, 2) == 0, xx, 1 - xx)

        def plane_idx(pp, zz):
            xq = pp // 4
            yq = jnp.where(xq == 0, pp, 7 - pp)
            return idx(xq, yq, zz)

        pf = plane_idx(lax.rem(p_ + 1, 8), z_)
        pb = plane_idx(lax.rem(p_ + 7, 8), z_)
        zf = idx(x_, y_, lax.rem(z_ + 1, 4))
        zb = idx(x_, y_, lax.rem(z_ + 3, 4))

        barrier = pltpu.get_barrier_semaphore()
        for nbr in (pf, pb, zf, zb):
            pl.semaphore_signal(barrier, inc=1, device_id=(nbr,),
                                device_id_type=MESH)
        pl.semaphore_wait(barrier, 4)

        wA = w_ref[:, 0:hc]
        wB = w_ref[:, hc:n]

        def remote(src, dst, ss, rr, dev):
            return pltpu.make_async_remote_copy(
                src_ref=src, dst_ref=dst, send_sem=ss, recv_sem=rr,
                device_id=(dev,), device_id_type=MESH)

        def bidir_rs(pos, size, partial_fn, stA, stB, bufA, bufB,
                     ssA, rrA, ssB, rrB, fwd, bwd):
            def md(a):
                return lax.rem(a, size)
            stA[0] = partial_fn(md(pos + size - 1), 0).astype(bf16)
            stB[0] = partial_fn(md(pos + 1), 1).astype(bf16)
            accA = accB = None
            for t in range(size - 1):
                rdA = remote(stA.at[t % 2], bufA.at[t], ssA.at[t], rrA.at[t], fwd)
                rdB = remote(stB.at[t % 2], bufB.at[t], ssB.at[t], rrB.at[t], bwd)
                rdA.start()
                rdB.start()
                pA = partial_fn(md(pos + 2 * size - 2 - t), 0)
                pB = partial_fn(md(pos + 2 + t), 1)
                rdA.wait_recv()
                accA = pA + bufA[t].astype(f32)
                if t < size - 2:
                    stA[(t + 1) % 2] = accA.astype(bf16)
                rdB.wait_recv()
                accB = pB + bufB[t].astype(f32)
                if t < size - 2:
                    stB[(t + 1) % 2] = accB.astype(bf16)
                rdA.wait_send()
                rdB.wait_send()
            return accA, accB

        def bidir_ag(pos, size, ownA_, ownB_, agA, agB, ssA, rrA, ssB, rrB,
                     fwd, bwd, store):
            def md(a):
                return lax.rem(a, size)
            for h in range(size - 1):
                srcA = ownA_ if h == 0 else agA.at[h - 1]
                srcB = ownB_ if h == 0 else agB.at[h - 1]
                rdA = remote(srcA, agA.at[h], ssA.at[h], rrA.at[h], fwd)
                rdB = remote(srcB, agB.at[h], ssB.at[h], rrB.at[h], bwd)
                rdA.start()
                rdB.start()
                rdA.wait_recv()
                store(0, md(pos + size - 1 - h), agA[h])
                rdB.wait_recv()
                store(1, md(pos + 1 + h), agB[h])
                rdA.wait_send()
                rdB.wait_send()

        def pplane(g, half):
            xs = x_ref[pl.ds(g * rp, rp), :]
            return jnp.dot(xs, wA if half == 0 else wB,
                           preferred_element_type=f32)

        pA_, pB_ = bidir_rs(p_, 8, pplane, pstA, pstB, prsA, prsB,
                            pssA, prrA, pssB, prrB, pf, pb)
        planeA[:, :] = pA_
        planeB[:, :] = pB_

        def pzc(g, half):
            acc = planeA if half == 0 else planeB
            return acc[pl.ds(g * rz, rz), :]

        fA, fB = bidir_rs(z_, 4, pzc, zstA, zstB, zrsA, zrsB,
                          zssA, zrrA, zssB, zrrB, zf, zb)

        g64 = _gelu(jnp.concatenate([fA, fB], axis=1))
        r0 = rp * p_ + rz * z_
        out_ref[pl.ds(r0, rz), :] = g64
        own_buf[:, :] = g64.astype(bf16)
        ownA[:, :] = own_buf[:, 0:hc]
        ownB[:, :] = own_buf[:, hc:n]
        psrcA[pl.ds(rz * z_, rz), :] = ownA[:, :]
        psrcB[pl.ds(rz * z_, rz), :] = ownB[:, :]

        def store_zag(half, g, data):
            row = rp * p_ + rz * g
            if half == 0:
                out_ref[pl.ds(row, rz), 0:hc] = data.astype(f32)
                psrcA[pl.ds(rz * g, rz), :] = data
            else:
                out_ref[pl.ds(row, rz), hc:n] = data.astype(f32)
                psrcB[pl.ds(rz * g, rz), :] = data

        bidir_ag(z_, 4, ownA, ownB, zagA, zagB,
                 gzssA, gzrrA, gzssB, gzrrB, zf, zb, store_zag)

        def store_pag(half, g, data):
            if half == 0:
                out_ref[pl.ds(rp * g, rp), 0:hc] = data.astype(f32)
            else:
                out_ref[pl.ds(rp * g, rp), hc:n] = data.astype(f32)

        bidir_ag(p_, 8, psrcA, psrcB, pagA, pagB,
                 gpssA, gprrA, gpssB, gprrB, pf, pb, store_pag)

    dma7 = pltpu.SemaphoreType.DMA((7,))
    dma3 = pltpu.SemaphoreType.DMA((3,))
    return pl.pallas_call(
        body,
        out_shape=jax.ShapeDtypeStruct((m, n), jnp.float32),
        in_specs=[
            pl.BlockSpec(memory_space=pltpu.VMEM),
            pl.BlockSpec(memory_space=pltpu.VMEM),
        ],
        out_specs=pl.BlockSpec(memory_space=pltpu.VMEM),
        scratch_shapes=[
            pltpu.VMEM((2, rp, hc), bf16),
            pltpu.VMEM((2, rp, hc), bf16),
            pltpu.VMEM((7, rp, hc), bf16),
            pltpu.VMEM((7, rp, hc), bf16),
            pltpu.VMEM((2, rz, hc), bf16),
            pltpu.VMEM((2, rz, hc), bf16),
            pltpu.VMEM((3, rz, hc), bf16),
            pltpu.VMEM((3, rz, hc), bf16),
            pltpu.VMEM((rp, hc), f32),
            pltpu.VMEM((rp, hc), f32),
            pltpu.VMEM((rz, n), bf16),
            pltpu.VMEM((rz, hc), bf16),
            pltpu.VMEM((rz, hc), bf16),
            pltpu.VMEM((3, rz, hc), bf16),
            pltpu.VMEM((3, rz, hc), bf16),
            pltpu.VMEM((rp, hc), bf16),
            pltpu.VMEM((rp, hc), bf16),
            pltpu.VMEM((7, rp, hc), bf16),
            pltpu.VMEM((7, rp, hc), bf16),
            dma7, dma7, dma7, dma7,
            dma3, dma3, dma3, dma3,
            dma3, dma3, dma3, dma3,
            dma7, dma7, dma7, dma7,
        ],
        compiler_params=pltpu.CompilerParams(
            collective_id=0, vmem_limit_bytes=64 * 1024 * 1024),
    )(x, w_mat)
